# baseline (speedup 1.0000x reference)
import functools

import jax
import jax.numpy as jnp
from jax import lax
from jax.experimental import pallas as pl
from jax.experimental.pallas import tpu as pltpu
from jax.experimental.pallas import tpu_sc as plsc

_NUM_METRICS = 10
_NUM_TASKS = 8
_NUM_ACTIONS = 1024
_B = 16384
_LANES = 16
_NC = 2
_NS = 16
_NW = _NC * _NS
_RPW = _B // _NW
_GROUPS = _RPW // _LANES
_NCOLS = _NUM_METRICS + _NUM_TASKS
_TABLE_PAD = 80

_OUT_SHAPE = jax.ShapeDtypeStruct((_B, _NUM_ACTIONS), jnp.float32)
_FLAT_SHAPE = jax.ShapeDtypeStruct((_B * _NUM_ACTIONS,), jnp.float32)
_NCHUNK = _RPW // 128


def _make_sc_scatter():
    mesh = plsc.VectorSubcoreMesh(core_axis_name="c", subcore_axis_name="s")

    @functools.partial(
        pl.kernel,
        mesh=mesh,
        out_type=(),
        scratch_types=[
            pltpu.VMEM((_RPW, 128), jnp.float32),
            pltpu.VMEM((_TABLE_PAD,), jnp.int32),
            pltpu.VMEM((_NCHUNK, 128), jnp.int32),
            pltpu.VMEM((128,), jnp.float32),
            pltpu.VMEM((128,), jnp.float32),
            pltpu.SemaphoreType.DMA,
        ],
    )
    def sc_scatter(state_hbm, table_hbm, p_out, l_out, p2_out,
                   sbuf, tbuf, fbuf, ones, zeros, sem):
        wid = lax.axis_index("s") * _NC + lax.axis_index("c")
        base = wid * _RPW
        pltpu.sync_copy(state_hbm.at[pl.ds(base, _RPW), pl.ds(0, 128)],
                        sbuf)
        pltpu.sync_copy(table_hbm, tbuf)
        for q in range(128 // _LANES):
            ones[pl.ds(q * _LANES, _LANES)] = jnp.full(
                (_LANES,), 1.0, jnp.float32)
            zeros[pl.ds(q * _LANES, _LANES)] = jnp.zeros(
                (_LANES,), jnp.float32)

        tvecs = [tbuf[pl.ds(h * _LANES, _LANES)]
                 for h in range(_TABLE_PAD // _LANES)]

        lane = lax.iota(jnp.int32, _LANES)
        zeros_i = jnp.zeros((_LANES,), jnp.int32)
        ninf = jnp.full((_LANES,), -jnp.inf, jnp.float32)
        gdn = lax.GatherDimensionNumbers(
            offset_dims=(), collapsed_slice_dims=(0,), start_index_map=(0,))

        def vgather(vec, idx):
            return lax.gather(
                vec, idx[:, None], gdn, slice_sizes=(1,),
                mode=lax.GatherScatterMode.PROMISE_IN_BOUNDS)

        def hmax(v):
            for k in (8, 4, 2, 1):
                v = jnp.maximum(v, vgather(v, lax.bitwise_and(lane + k, 15)))
            return v

        def hmin(v):
            for k in (8, 4, 2, 1):
                v = jnp.minimum(v, vgather(v, lax.bitwise_and(lane + k, 15)))
            return v

        big = jnp.full((_LANES,), 16, jnp.int32)

        def hargmax(v):
            eq = v == hmax(v)
            return hmin(jnp.where(eq, lane, big))

        def row_body(r, acc):
            v0 = sbuf[r, pl.ds(0, _LANES)]
            v1 = sbuf[r, pl.ds(_LANES, _LANES)]
            mvec = jnp.where((lane >= 1) & (lane <= _NUM_METRICS), v0, ninf)
            metric = hargmax(mvec) - 1
            p1 = vgather(v0, jnp.minimum(lane + 11, 15))
            p2 = vgather(v1, jnp.maximum(lane - 5, 0))
            tvec = jnp.where(lane < 5, p1, p2)
            tvec = jnp.where(lane < _NUM_TASKS, tvec, ninf)
            task = hargmax(tvec)

            code = zeros_i + task * _NUM_METRICS + metric
            high = lax.shift_right_logical(code, 4)
            low = lax.bitwise_and(code, jnp.full((_LANES,), 15, jnp.int32))
            aidx = zeros_i
            for h, tvec_t in enumerate(tvecs):
                aidx = jnp.where(high == h, vgather(tvec_t, low), aidx)

            fi = (base + r) * _NUM_ACTIONS + aidx
            acc = jnp.where(lane == lax.bitwise_and(r, 15), fi, acc)

            @pl.when(lax.bitwise_and(r, 15) == 15)
            def _():
                g = lax.div(r, 16)
                j = lax.div(g, 128 // _LANES)
                o = lax.rem(g, 128 // _LANES) * _LANES
                fbuf[j, pl.ds(o, _LANES)] = acc

            return acc

        lax.fori_loop(0, _RPW, row_body, zeros_i)

        copies = []
        for j in range(_NCHUNK):
            idx = fbuf.at[j]
            copies.append(pltpu.async_copy(ones, p_out.at[idx], sem))
            copies.append(pltpu.async_copy(ones, p2_out.at[idx], sem))
            copies.append(pltpu.async_copy(zeros, l_out.at[idx], sem))
        for c in copies:
            c.wait()

    return sc_scatter


_sc_scatter_cache = []


def _get_sc_scatter():
    if not _sc_scatter_cache:
        _sc_scatter_cache.append(_make_sc_scatter())
    return _sc_scatter_cache[0]


_R = 512
_G = _B // _R


_RF = _R * _NUM_ACTIONS


def _tc_fill_body(probs_ref, logits_ref, probs2_ref):
    probs_ref[...] = jnp.zeros((_RF,), jnp.float32)
    probs2_ref[...] = jnp.zeros((_RF,), jnp.float32)
    logits_ref[...] = jnp.full((_RF,), -1000000.0, jnp.float32)


_tc_fill = pl.pallas_call(
    _tc_fill_body,
    grid=(_G,),
    in_specs=[],
    out_specs=[
        pl.BlockSpec((_RF,), lambda i: (i,)),
        pl.BlockSpec((_RF,), lambda i: (i,)),
        pl.BlockSpec((_RF,), lambda i: (i,)),
    ],
    out_shape=[_FLAT_SHAPE, _FLAT_SHAPE, _FLAT_SHAPE],
)


def kernel(state, action_table):
    s = state.astype(jnp.float32)
    table = action_table.reshape(-1).astype(jnp.int32)
    p0, l0, p20 = _tc_fill()
    p_ref = jax.new_ref(p0)
    l_ref = jax.new_ref(l0)
    p2_ref = jax.new_ref(p20)
    _get_sc_scatter()(s, table, p_ref, l_ref, p2_ref)
    probs = p_ref[...].reshape(_B, _NUM_ACTIONS)
    logits = l_ref[...].reshape(_B, _NUM_ACTIONS)
    probs2 = p2_ref[...].reshape(_B, _NUM_ACTIONS)
    fv = jnp.zeros((_B, 1), jnp.float32)
    return (probs, logits, probs2, fv)


# speedup vs baseline: 3.6272x; 3.6272x over previous
import functools

import jax
import jax.numpy as jnp
from jax import lax
from jax.experimental import pallas as pl
from jax.experimental.pallas import tpu as pltpu
from jax.experimental.pallas import tpu_sc as plsc

_NUM_METRICS = 10
_NUM_TASKS = 8
_NUM_ACTIONS = 1024
_B = 16384
_LANES = 16
_NC = 2
_NS = 16
_NW = _NC * _NS
_RPW = _B // _NW
_GROUPS = _RPW // _LANES
_NCOLS = _NUM_METRICS + _NUM_TASKS
_TABLE_PAD = 80


def _make_sc_action():
    mesh = plsc.VectorSubcoreMesh(core_axis_name="c", subcore_axis_name="s")

    @functools.partial(
        pl.kernel,
        mesh=mesh,
        out_type=jax.ShapeDtypeStruct((_B,), jnp.int32),
        scratch_types=[
            pltpu.VMEM((_NCOLS * _RPW,), jnp.float32),
            pltpu.VMEM((_TABLE_PAD,), jnp.int32),
            pltpu.VMEM((_RPW,), jnp.int32),
        ],
    )
    def sc_action(colsw_hbm, table_hbm, out_hbm, sbuf, tbuf, obuf):
        wid = lax.axis_index("s") * _NC + lax.axis_index("c")
        base = wid * _RPW
        pltpu.sync_copy(
            colsw_hbm.at[pl.ds(wid * _NCOLS * _RPW, _NCOLS * _RPW)], sbuf)
        pltpu.sync_copy(table_hbm, tbuf)

        tvecs = [tbuf[pl.ds(h * _LANES, _LANES)]
                 for h in range(_TABLE_PAD // _LANES)]

        zeros_i = jnp.zeros((_LANES,), jnp.int32)

        def group_body(g, carry):
            r0 = g * _LANES

            def col(c):
                return sbuf[pl.ds(c * _RPW + r0, _LANES)]

            def amax_step(k, c, off):
                bv, bi = c
                v = col(off + k)
                upd = v > bv
                return (jnp.where(upd, v, bv),
                        jnp.where(upd, zeros_i + k, bi))

            metric = lax.fori_loop(
                1, _NUM_METRICS,
                lambda k, c: amax_step(k, c, 0),
                (col(0), zeros_i))[1]

            ti = lax.fori_loop(
                1, _NUM_TASKS,
                lambda k, c: amax_step(k, c, _NUM_METRICS),
                (col(_NUM_METRICS), zeros_i))[1]

            code = ti * _NUM_METRICS + metric
            high = lax.shift_right_logical(code, 4)
            low = lax.bitwise_and(code, jnp.full((_LANES,), 15, jnp.int32))
            aidx = jnp.zeros((_LANES,), jnp.int32)
            for h, tvec in enumerate(tvecs):
                g_h = lax.gather(
                    tvec, low[:, None],
                    lax.GatherDimensionNumbers(
                        offset_dims=(), collapsed_slice_dims=(0,),
                        start_index_map=(0,)),
                    slice_sizes=(1,),
                    mode=lax.GatherScatterMode.PROMISE_IN_BOUNDS)
                aidx = jnp.where(high == h, g_h, aidx)
            obuf[pl.ds(r0, _LANES)] = aidx
            return carry

        lax.fori_loop(0, _GROUPS, group_body, 0)
        pltpu.sync_copy(obuf, out_hbm.at[pl.ds(base, _RPW)])

    return sc_action


_sc_action_cache = []


def _get_sc_action():
    if not _sc_action_cache:
        _sc_action_cache.append(_make_sc_action())
    return _sc_action_cache[0]


_R = 512
_G = _B // _R


def _tc_body(a_ref, probs_ref, logits_ref, probs2_ref):
    aidx = a_ref[...]
    cols = lax.broadcasted_iota(jnp.int32, (_R, _NUM_ACTIONS), 1)
    onehot = cols == aidx[:, None]
    p = onehot.astype(jnp.float32)
    probs_ref[...] = p
    probs2_ref[...] = p
    logits_ref[...] = jnp.where(onehot, jnp.float32(0.0),
                                jnp.float32(-1000000.0))


_tc_call = pl.pallas_call(
    _tc_body,
    grid=(_G,),
    in_specs=[pl.BlockSpec((_R,), lambda i: (i,))],
    out_specs=[
        pl.BlockSpec((_R, _NUM_ACTIONS), lambda i: (i, 0)),
        pl.BlockSpec((_R, _NUM_ACTIONS), lambda i: (i, 0)),
        pl.BlockSpec((_R, _NUM_ACTIONS), lambda i: (i, 0)),
    ],
    out_shape=[
        jax.ShapeDtypeStruct((_B, _NUM_ACTIONS), jnp.float32),
        jax.ShapeDtypeStruct((_B, _NUM_ACTIONS), jnp.float32),
        jax.ShapeDtypeStruct((_B, _NUM_ACTIONS), jnp.float32),
    ],
)


def kernel(state, action_table):
    s = state.astype(jnp.float32)
    cols = (s[:, 1:1 + _NCOLS]
            .reshape(_NW, _RPW, _NCOLS)
            .transpose(0, 2, 1)
            .reshape(-1))
    table = action_table.reshape(-1).astype(jnp.int32)
    aidx = _get_sc_action()(cols, table)
    probs, logits, probs2 = _tc_call(aidx)
    fv = jnp.zeros((_B, 1), jnp.float32)
    return (probs, logits, probs2, fv)


# speedup vs baseline: 3.6583x; 1.0086x over previous
import functools

import jax
import jax.numpy as jnp
from jax import lax
from jax.experimental import pallas as pl
from jax.experimental.pallas import tpu as pltpu
from jax.experimental.pallas import tpu_sc as plsc

_NUM_METRICS = 10
_NUM_TASKS = 8
_NUM_ACTIONS = 1024
_B = 16384
_LANES = 16
_NC = 2
_NS = 16
_NW = _NC * _NS
_RPW = _B // _NW
_GROUPS = _RPW // _LANES
_NCOLS = _NUM_METRICS + _NUM_TASKS
_TABLE_PAD = 80


def _make_sc_action():
    mesh = plsc.VectorSubcoreMesh(core_axis_name="c", subcore_axis_name="s")

    @functools.partial(
        pl.kernel,
        mesh=mesh,
        out_type=jax.ShapeDtypeStruct((_B,), jnp.int32),
        scratch_types=[
            pltpu.VMEM((_NCOLS * _RPW,), jnp.float32),
            pltpu.VMEM((_TABLE_PAD,), jnp.int32),
            pltpu.VMEM((_RPW,), jnp.int32),
        ],
    )
    def sc_action(colsw_hbm, table_hbm, out_hbm, sbuf, tbuf, obuf):
        wid = lax.axis_index("s") * _NC + lax.axis_index("c")
        base = wid * _RPW
        pltpu.sync_copy(
            colsw_hbm.at[pl.ds(wid * _NCOLS * _RPW, _NCOLS * _RPW)], sbuf)
        pltpu.sync_copy(table_hbm, tbuf)

        tvecs = [tbuf[pl.ds(h * _LANES, _LANES)]
                 for h in range(_TABLE_PAD // _LANES)]

        def group_body(g, carry):
            r0 = g * _LANES

            def col(c):
                return sbuf[pl.ds(c * _RPW + r0, _LANES)]

            bv = col(0)
            bi = jnp.zeros((_LANES,), jnp.int32)
            for k in range(1, _NUM_METRICS):
                v = col(k)
                upd = v > bv
                bv = jnp.where(upd, v, bv)
                bi = jnp.where(upd, jnp.full((_LANES,), k, jnp.int32), bi)
            metric = bi

            tv = col(_NUM_METRICS)
            ti = jnp.zeros((_LANES,), jnp.int32)
            for k in range(1, _NUM_TASKS):
                v = col(_NUM_METRICS + k)
                upd = v > tv
                tv = jnp.where(upd, v, tv)
                ti = jnp.where(upd, jnp.full((_LANES,), k, jnp.int32), ti)

            code = ti * _NUM_METRICS + metric
            high = lax.shift_right_logical(code, 4)
            low = lax.bitwise_and(code, jnp.full((_LANES,), 15, jnp.int32))
            aidx = jnp.zeros((_LANES,), jnp.int32)
            for h, tvec in enumerate(tvecs):
                g_h = lax.gather(
                    tvec, low[:, None],
                    lax.GatherDimensionNumbers(
                        offset_dims=(), collapsed_slice_dims=(0,),
                        start_index_map=(0,)),
                    slice_sizes=(1,),
                    mode=lax.GatherScatterMode.PROMISE_IN_BOUNDS)
                aidx = jnp.where(high == h, g_h, aidx)
            obuf[pl.ds(r0, _LANES)] = aidx
            return carry

        lax.fori_loop(0, _GROUPS, group_body, 0)
        pltpu.sync_copy(obuf, out_hbm.at[pl.ds(base, _RPW)])

    return sc_action


_sc_action_cache = []


def _get_sc_action():
    if not _sc_action_cache:
        _sc_action_cache.append(_make_sc_action())
    return _sc_action_cache[0]


_R = 512
_G = _B // _R


def _tc_body(a_ref, probs_ref, logits_ref, probs2_ref):
    aidx = a_ref[...]
    cols = lax.broadcasted_iota(jnp.int32, (_R, _NUM_ACTIONS), 1)
    onehot = cols == aidx[:, None]
    p = onehot.astype(jnp.float32)
    probs_ref[...] = p
    probs2_ref[...] = p
    logits_ref[...] = jnp.where(onehot, jnp.float32(0.0),
                                jnp.float32(-1000000.0))


_tc_call = pl.pallas_call(
    _tc_body,
    grid=(_G,),
    in_specs=[pl.BlockSpec((_R,), lambda i: (i,))],
    out_specs=[
        pl.BlockSpec((_R, _NUM_ACTIONS), lambda i: (i, 0)),
        pl.BlockSpec((_R, _NUM_ACTIONS), lambda i: (i, 0)),
        pl.BlockSpec((_R, _NUM_ACTIONS), lambda i: (i, 0)),
    ],
    out_shape=[
        jax.ShapeDtypeStruct((_B, _NUM_ACTIONS), jnp.float32),
        jax.ShapeDtypeStruct((_B, _NUM_ACTIONS), jnp.float32),
        jax.ShapeDtypeStruct((_B, _NUM_ACTIONS), jnp.float32),
    ],
)


def kernel(state, action_table):
    s = state.astype(jnp.float32)
    cols = (s[:, 1:1 + _NCOLS]
            .reshape(_NW, _RPW, _NCOLS)
            .transpose(0, 2, 1)
            .reshape(-1))
    table = action_table.reshape(-1).astype(jnp.int32)
    aidx = _get_sc_action()(cols, table)
    probs, logits, probs2 = _tc_call(aidx)
    fv = jnp.zeros((_B, 1), jnp.float32)
    return (probs, logits, probs2, fv)
